# baseline (speedup 1.0000x reference)
import functools

import jax
import jax.numpy as jnp
from jax import lax
from jax.experimental import pallas as pl
from jax.experimental.pallas import tpu as pltpu
from jax.experimental.pallas import tpu_sc as plsc

N_NODES = 10000
N_EDGES = 320000
D = 128

NC = 2
NS = 16
NW = NC * NS
CHUNK = 64
CPW = 160
NWIN = 4
CPW_W = CPW // NWIN
NSLOT = 4
LOOK = 2
PEEL = 2
INNER = 4
NMAIN = 9
TAIL = CPW_W - PEEL - NMAIN * INNER
EPW = CPW * CHUNK
E_PAD = NW * EPW
N_PAD = 10240
RPT = N_PAD // NS
BR = 1000
GRID = N_NODES // BR


def _make_sc_agg(compute_cnt):
    mesh = plsc.VectorSubcoreMesh(core_axis_name="c", subcore_axis_name="s")
    out_type = [jax.ShapeDtypeStruct((NC, N_PAD, D), jnp.float32)]
    if compute_cnt:
        out_type.append(jax.ShapeDtypeStruct((NC, N_PAD), jnp.float32))

    scratch = [
        pltpu.VMEM((CPW_W, CHUNK), jnp.int32),
        pltpu.VMEM((CPW_W, CHUNK), jnp.int32),
        pltpu.VMEM((NSLOT, CHUNK, D), jnp.float32),
        pltpu.VMEM((CHUNK,), jnp.float32),
        pltpu.VMEM_SHARED((N_PAD, D), jnp.float32),
        pltpu.VMEM_SHARED((N_PAD,), jnp.float32),
    ] + [pltpu.SemaphoreType.DMA] * (2 * NSLOT + 1)

    def body(x_hbm, src_hbm, dst_hbm, zrow_hbm, zcnt_hbm, *rest):
        if compute_cnt:
            out_hbm, cnt_hbm = rest[0], rest[1]
            refs = rest[2:]
        else:
            out_hbm = rest[0]
            refs = rest[1:]
        src_v, dst_v, rows_v, ones_v, agg_sh, cnt_sh = refs[:6]
        gsems = refs[6:6 + NSLOT]
        ssems = refs[6 + NSLOT:6 + 2 * NSLOT]
        csem = refs[6 + 2 * NSLOT]
        c = lax.axis_index("c")
        s = lax.axis_index("s")
        wid = c * NS + s
        rb = s * RPT
        pltpu.sync_copy(zrow_hbm.at[pl.ds(rb, RPT)], agg_sh.at[pl.ds(rb, RPT)])
        if compute_cnt:
            pltpu.sync_copy(zcnt_hbm.at[pl.ds(rb, RPT)],
                            cnt_sh.at[pl.ds(rb, RPT)])
            for i in range(CHUNK // 16):
                ones_v[pl.ds(i * 16, 16)] = jnp.full((16,), 1.0, jnp.float32)
        plsc.subcore_barrier()

        def gissue(j, b):
            pltpu.async_copy(x_hbm.at[src_v.at[j]], rows_v.at[b], gsems[b])

        def gwait(b):
            pltpu.make_async_copy(x_hbm.at[src_v.at[0]], rows_v.at[b],
                                  gsems[b]).wait()

        def sissue(j, b):
            pltpu.async_copy(rows_v.at[b], agg_sh.at[dst_v.at[j]], ssems[b],
                             add=True)
            if compute_cnt:
                pltpu.async_copy(ones_v, cnt_sh.at[dst_v.at[j]], csem,
                                 add=True)

        def swait(b):
            pltpu.make_async_copy(rows_v.at[b], agg_sh.at[dst_v.at[0]],
                                  ssems[b]).wait()

        for win in range(NWIN):
            pltpu.sync_copy(src_hbm.at[wid, pl.ds(win * CPW_W, CPW_W)],
                            src_v)
            pltpu.sync_copy(dst_hbm.at[wid, pl.ds(win * CPW_W, CPW_W)],
                            dst_v)
            for k in range(LOOK):
                gissue(k, k)
            for i in range(PEEL):
                gwait(i % NSLOT)
                sissue(i, i % NSLOT)
                gissue(i + LOOK, (i + LOOK) % NSLOT)

            def outer_body(o, carry):
                for jj in range(INNER):
                    j = o * INNER + jj + PEEL
                    b = (jj + PEEL) % NSLOT
                    bg = (jj + PEEL + LOOK) % NSLOT
                    gwait(b)
                    sissue(j, b)
                    swait(bg)
                    gissue(j + LOOK, bg)
                return carry

            lax.fori_loop(0, NMAIN, outer_body, 0)
            for i in range(TAIL):
                j = PEEL + NMAIN * INNER + i
                gwait(j % NSLOT)
                sissue(j, j % NSLOT)
            for b in range(NSLOT):
                swait(b)
            if compute_cnt:
                for _ in range(CPW_W):
                    pltpu.make_async_copy(ones_v, cnt_sh.at[dst_v.at[0]],
                                          csem).wait()
        plsc.subcore_barrier()
        pltpu.sync_copy(agg_sh.at[pl.ds(rb, RPT)], out_hbm.at[c, pl.ds(rb, RPT)])
        if compute_cnt:
            pltpu.sync_copy(cnt_sh.at[pl.ds(rb, RPT)],
                            cnt_hbm.at[c, pl.ds(rb, RPT)])

    return pl.kernel(body, out_type=out_type, mesh=mesh,
                     scratch_types=scratch)


_sc_agg_cnt = _make_sc_agg(True)
_sc_agg = _make_sc_agg(False)


_row_spec = pl.BlockSpec((BR, D), lambda i: (i, 0))
_cnt_spec = pl.BlockSpec((BR, 1), lambda i: (i, 0))
_mat_spec = pl.BlockSpec((D, D), lambda i: (0, 0))
_bias_spec = pl.BlockSpec((1, D), lambda i: (0, 0))
_out_shape = jax.ShapeDtypeStruct((N_NODES, D), jnp.float32)


def _dense_r_body(x, wr, b, o):
    o[...] = lax.dot_general(x[...], wr[...], (((1,), (1,)), ((), ())),
                             preferred_element_type=jnp.float32) + b[...]


def _dense_r(x, wr, b):
    return pl.pallas_call(
        _dense_r_body, grid=(GRID,),
        in_specs=[_row_spec, _mat_spec, _bias_spec],
        out_specs=_row_spec, out_shape=_out_shape,
    )(x, wr, b)


def _dense_l_body(relu, p0, p1, c0, c1, xr, wl, o):
    cnt = jnp.maximum(c0[...] + c1[...], 1.0)
    mean = (p0[...] + p1[...]) / cnt
    h = lax.dot_general(mean, wl[...], (((1,), (1,)), ((), ())),
                        preferred_element_type=jnp.float32) + xr[...]
    if relu:
        h = jnp.maximum(h, 0.0)
    o[...] = h


def _dense_l(relu, p0, p1, c0, c1, xr, wl):
    return pl.pallas_call(
        functools.partial(_dense_l_body, relu),
        grid=(GRID,),
        in_specs=[_row_spec, _row_spec, _cnt_spec, _cnt_spec, _row_spec,
                  _mat_spec],
        out_specs=_row_spec, out_shape=_out_shape,
    )(p0, p1, c0, c1, xr, wl)


def kernel(x, edge_index, Wl1, Wr1, b1, Wl2, Wr2, b2):
    src = edge_index[0].astype(jnp.int32)
    dst = edge_index[1].astype(jnp.int32)
    pad = E_PAD - N_EDGES
    ar = jnp.arange(pad, dtype=jnp.int32)
    src_p = jnp.concatenate([src, ar % N_NODES]).reshape(NW, CPW, CHUNK)
    dst_p = jnp.concatenate([dst, N_NODES + ar % (N_PAD - N_NODES)]
                            ).reshape(NW, CPW, CHUNK)
    zrow = jnp.zeros((N_PAD, D), jnp.float32)
    zcnt = jnp.zeros((N_PAD,), jnp.float32)

    agg1, cnt = _sc_agg_cnt(x, src_p, dst_p, zrow, zcnt)
    xr1 = _dense_r(x, Wr1, b1.reshape(1, D))
    c0 = cnt[0].reshape(N_PAD, 1)
    c1 = cnt[1].reshape(N_PAD, 1)
    h = _dense_l(True, agg1[0], agg1[1], c0, c1, xr1, Wl1)
    (agg2,) = _sc_agg(h, src_p, dst_p, zrow, zcnt)
    hr2 = _dense_r(h, Wr2, b2.reshape(1, D))
    out = _dense_l(False, agg2[0], agg2[1], c0, c1, hr2, Wl2)
    return out


# speedup vs baseline: 1.0982x; 1.0982x over previous
import functools

import jax
import jax.numpy as jnp
from jax import lax
from jax.experimental import pallas as pl
from jax.experimental.pallas import tpu as pltpu
from jax.experimental.pallas import tpu_sc as plsc

N_NODES = 10000
N_EDGES = 320000
D = 128

NC = 2
NS = 16
NW = NC * NS
CHUNK = 128
CPW = 80
CPW_H = CPW // 2
EPW = CPW * CHUNK
E_PAD = NW * EPW
N_PAD = 10240
RPT = N_PAD // NS
BR = 1000
GRID = N_NODES // BR


def _make_sc_agg(compute_cnt):
    mesh = plsc.VectorSubcoreMesh(core_axis_name="c", subcore_axis_name="s")
    out_type = [jax.ShapeDtypeStruct((NC, N_PAD, D), jnp.float32)]
    if compute_cnt:
        out_type.append(jax.ShapeDtypeStruct((NC, N_PAD), jnp.float32))

    scratch = [
        pltpu.VMEM((CPW_H, CHUNK), jnp.int32),
        pltpu.VMEM((CPW_H, CHUNK), jnp.int32),
        pltpu.VMEM((2, CHUNK, D), jnp.float32),
        pltpu.VMEM((CHUNK,), jnp.float32),
        pltpu.VMEM_SHARED((N_PAD, D), jnp.float32),
        pltpu.VMEM_SHARED((N_PAD,), jnp.float32),
        pltpu.SemaphoreType.DMA,
        pltpu.SemaphoreType.DMA,
    ]

    def body(x_hbm, src_hbm, dst_hbm, zrow_hbm, zcnt_hbm, *rest):
        if compute_cnt:
            out_hbm, cnt_hbm = rest[0], rest[1]
            refs = rest[2:]
        else:
            out_hbm = rest[0]
            refs = rest[1:]
        src_v, dst_v, rows_v, ones_v, agg_sh, cnt_sh, sem0, sem1 = refs
        sems = (sem0, sem1)
        c = lax.axis_index("c")
        s = lax.axis_index("s")
        wid = c * NS + s
        rb = s * RPT
        pltpu.sync_copy(zrow_hbm.at[pl.ds(rb, RPT)], agg_sh.at[pl.ds(rb, RPT)])
        if compute_cnt:
            pltpu.sync_copy(zcnt_hbm.at[pl.ds(rb, RPT)],
                            cnt_sh.at[pl.ds(rb, RPT)])
            for i in range(CHUNK // 16):
                ones_v[pl.ds(i * 16, 16)] = jnp.full((16,), 1.0, jnp.float32)
        plsc.subcore_barrier()

        INNER = 10

        for half in range(2):
            pltpu.sync_copy(src_hbm.at[wid, pl.ds(half * CPW_H, CPW_H)],
                            src_v)
            pltpu.sync_copy(dst_hbm.at[wid, pl.ds(half * CPW_H, CPW_H)],
                            dst_v)
            for b in range(2):
                pltpu.async_copy(x_hbm.at[src_v.at[b]], rows_v.at[b],
                                 sems[b])

            def outer_body(o, carry):
                for jj in range(INNER):
                    j = o * INNER + jj
                    b = jj % 2
                    pltpu.make_async_copy(x_hbm.at[src_v.at[j]],
                                          rows_v.at[b], sems[b]).wait()
                    pltpu.sync_copy(rows_v.at[b], agg_sh.at[dst_v.at[j]],
                                    add=True)
                    if compute_cnt:
                        pltpu.sync_copy(ones_v, cnt_sh.at[dst_v.at[j]],
                                        add=True)
                    jn = jnp.minimum(j + 2, CPW_H - 1)
                    pltpu.async_copy(x_hbm.at[src_v.at[jn]], rows_v.at[b],
                                     sems[b])
                return carry

            lax.fori_loop(0, CPW_H // INNER, outer_body, 0)
            for b in range(2):
                pltpu.make_async_copy(x_hbm.at[src_v.at[0]], rows_v.at[b],
                                      sems[b]).wait()
        plsc.subcore_barrier()
        pltpu.sync_copy(agg_sh.at[pl.ds(rb, RPT)], out_hbm.at[c, pl.ds(rb, RPT)])
        if compute_cnt:
            pltpu.sync_copy(cnt_sh.at[pl.ds(rb, RPT)],
                            cnt_hbm.at[c, pl.ds(rb, RPT)])

    return pl.kernel(body, out_type=out_type, mesh=mesh,
                     scratch_types=scratch)


_sc_agg_cnt = _make_sc_agg(True)
_sc_agg = _make_sc_agg(False)


_row_spec = pl.BlockSpec((BR, D), lambda i: (i, 0))
_cnt_spec = pl.BlockSpec((BR, 1), lambda i: (i, 0))
_mat_spec = pl.BlockSpec((D, D), lambda i: (0, 0))
_bias_spec = pl.BlockSpec((1, D), lambda i: (0, 0))
_out_shape = jax.ShapeDtypeStruct((N_NODES, D), jnp.float32)


def _dense_body(relu, p0, p1, c0, c1, x, wl, wr, b, o):
    cnt = jnp.maximum(c0[...] + c1[...], 1.0)
    mean = (p0[...] + p1[...]) / cnt
    h = (lax.dot_general(mean, wl[...], (((1,), (1,)), ((), ())),
                         preferred_element_type=jnp.float32)
         + lax.dot_general(x[...], wr[...], (((1,), (1,)), ((), ())),
                           preferred_element_type=jnp.float32)
         + b[...])
    if relu:
        h = jnp.maximum(h, 0.0)
    o[...] = h


def _dense(relu, p0, p1, c0, c1, x, wl, wr, b):
    return pl.pallas_call(
        functools.partial(_dense_body, relu),
        grid=(GRID,),
        in_specs=[_row_spec, _row_spec, _cnt_spec, _cnt_spec, _row_spec,
                  _mat_spec, _mat_spec, _bias_spec],
        out_specs=_row_spec, out_shape=_out_shape,
    )(p0, p1, c0, c1, x, wl, wr, b)


def kernel(x, edge_index, Wl1, Wr1, b1, Wl2, Wr2, b2):
    src = edge_index[0].astype(jnp.int32)
    dst = edge_index[1].astype(jnp.int32)
    pad = E_PAD - N_EDGES
    ar = jnp.arange(pad, dtype=jnp.int32)
    src_p = jnp.concatenate([src, ar % N_NODES]).reshape(NW, CPW, CHUNK)
    dst_p = jnp.concatenate([dst, N_NODES + ar % (N_PAD - N_NODES)]
                            ).reshape(NW, CPW, CHUNK)
    zrow = jnp.zeros((N_PAD, D), jnp.float32)
    zcnt = jnp.zeros((N_PAD,), jnp.float32)

    agg1, cnt = _sc_agg_cnt(x, src_p, dst_p, zrow, zcnt)
    c0 = cnt[0].reshape(N_PAD, 1)
    c1 = cnt[1].reshape(N_PAD, 1)
    h = _dense(True, agg1[0], agg1[1], c0, c1, x, Wl1, Wr1,
               b1.reshape(1, D))
    (agg2,) = _sc_agg(h, src_p, dst_p, zrow, zcnt)
    out = _dense(False, agg2[0], agg2[1], c0, c1, h, Wl2, Wr2,
                 b2.reshape(1, D))
    return out


# speedup vs baseline: 1.1333x; 1.0320x over previous
import functools

import jax
import jax.numpy as jnp
from jax import lax
from jax.experimental import pallas as pl
from jax.experimental.pallas import tpu as pltpu
from jax.experimental.pallas import tpu_sc as plsc

N_NODES = 10000
N_EDGES = 320000
D = 128

NC = 2
NS = 16
NW = NC * NS
CHUNK = 128
CPW = 80
CPW_H = CPW // 2
EPW = CPW * CHUNK
E_PAD = NW * EPW
N_PAD = 10240
RPT = N_PAD // NS
BR = 1000
GRID = N_NODES // BR


def _make_sc_agg(compute_cnt):
    mesh = plsc.VectorSubcoreMesh(core_axis_name="c", subcore_axis_name="s")
    out_type = [jax.ShapeDtypeStruct((NC, N_PAD, D), jnp.float32)]
    if compute_cnt:
        out_type.append(jax.ShapeDtypeStruct((NC, N_PAD), jnp.float32))

    scratch = [
        pltpu.VMEM((CPW_H, CHUNK), jnp.int32),
        pltpu.VMEM((CPW_H, CHUNK), jnp.int32),
        pltpu.VMEM((2, CHUNK, D), jnp.float32),
        pltpu.VMEM((CHUNK,), jnp.float32),
        pltpu.VMEM_SHARED((N_PAD, D), jnp.float32),
        pltpu.VMEM_SHARED((N_PAD,), jnp.float32),
        pltpu.SemaphoreType.DMA,
        pltpu.SemaphoreType.DMA,
        pltpu.SemaphoreType.DMA,
    ]

    def body(x_hbm, src_hbm, dst_hbm, *rest):
        if compute_cnt:
            out_hbm, cnt_hbm = rest[0], rest[1]
            refs = rest[2:]
        else:
            out_hbm = rest[0]
            refs = rest[1:]
        src_v, dst_v, rows_v, ones_v, agg_sh, cnt_sh, sem0, sem1, csem = refs
        sems = (sem0, sem1)
        c = lax.axis_index("c")
        s = lax.axis_index("s")
        wid = c * NS + s
        rb = s * RPT
        zv = jnp.zeros((16,), jnp.float32)

        def zrow_body(i, carry):
            for k in range(D // 16):
                rows_v[0, i, pl.ds(k * 16, 16)] = zv
            return carry

        lax.fori_loop(0, CHUNK, zrow_body, 0)
        for r in range(RPT // CHUNK):
            pltpu.sync_copy(rows_v.at[0],
                            agg_sh.at[pl.ds(rb + r * CHUNK, CHUNK)])
        if compute_cnt:
            for i in range(CHUNK // 16):
                ones_v[pl.ds(i * 16, 16)] = jnp.full((16,), 1.0, jnp.float32)
            for r in range(RPT // CHUNK):
                pltpu.sync_copy(rows_v.at[0, 0],
                                cnt_sh.at[pl.ds(rb + r * CHUNK, CHUNK)])
        plsc.subcore_barrier()

        INNER = 10

        for half in range(2):
            pltpu.sync_copy(src_hbm.at[wid, pl.ds(half * CPW_H, CPW_H)],
                            src_v)
            pltpu.sync_copy(dst_hbm.at[wid, pl.ds(half * CPW_H, CPW_H)],
                            dst_v)
            for b in range(2):
                pltpu.async_copy(x_hbm.at[src_v.at[b]], rows_v.at[b],
                                 sems[b])

            def outer_body(o, carry):
                for jj in range(INNER):
                    j = o * INNER + jj
                    b = jj % 2
                    pltpu.make_async_copy(x_hbm.at[src_v.at[j]],
                                          rows_v.at[b], sems[b]).wait()
                    pltpu.sync_copy(rows_v.at[b], agg_sh.at[dst_v.at[j]],
                                    add=True)
                    if compute_cnt:
                        pltpu.async_copy(ones_v, cnt_sh.at[dst_v.at[j]],
                                         csem, add=True)
                    jn = jnp.minimum(j + 2, CPW_H - 1)
                    pltpu.async_copy(x_hbm.at[src_v.at[jn]], rows_v.at[b],
                                     sems[b])
                return carry

            lax.fori_loop(0, CPW_H // INNER, outer_body, 0)
            for b in range(2):
                pltpu.make_async_copy(x_hbm.at[src_v.at[0]], rows_v.at[b],
                                      sems[b]).wait()
            if compute_cnt:
                for _ in range(CPW_H):
                    pltpu.make_async_copy(ones_v, cnt_sh.at[dst_v.at[0]],
                                          csem).wait()
        plsc.subcore_barrier()
        pltpu.sync_copy(agg_sh.at[pl.ds(rb, RPT)], out_hbm.at[c, pl.ds(rb, RPT)])
        if compute_cnt:
            pltpu.sync_copy(cnt_sh.at[pl.ds(rb, RPT)],
                            cnt_hbm.at[c, pl.ds(rb, RPT)])

    return pl.kernel(body, out_type=out_type, mesh=mesh,
                     scratch_types=scratch)


_sc_agg_cnt = _make_sc_agg(True)
_sc_agg = _make_sc_agg(False)


_row_spec = pl.BlockSpec((BR, D), lambda i: (i, 0))
_cnt_spec = pl.BlockSpec((BR, 1), lambda i: (i, 0))
_mat_spec = pl.BlockSpec((D, D), lambda i: (0, 0))
_bias_spec = pl.BlockSpec((1, D), lambda i: (0, 0))
_out_shape = jax.ShapeDtypeStruct((N_NODES, D), jnp.float32)


def _dense_body(relu, p0, p1, c0, c1, x, wl, wr, b, o):
    cnt = jnp.maximum(c0[...] + c1[...], 1.0)
    mean = (p0[...] + p1[...]) / cnt
    h = (lax.dot_general(mean, wl[...], (((1,), (1,)), ((), ())),
                         preferred_element_type=jnp.float32)
         + lax.dot_general(x[...], wr[...], (((1,), (1,)), ((), ())),
                           preferred_element_type=jnp.float32)
         + b[...])
    if relu:
        h = jnp.maximum(h, 0.0)
    o[...] = h


def _dense(relu, p0, p1, c0, c1, x, wl, wr, b):
    return pl.pallas_call(
        functools.partial(_dense_body, relu),
        grid=(GRID,),
        in_specs=[_row_spec, _row_spec, _cnt_spec, _cnt_spec, _row_spec,
                  _mat_spec, _mat_spec, _bias_spec],
        out_specs=_row_spec, out_shape=_out_shape,
    )(p0, p1, c0, c1, x, wl, wr, b)


def kernel(x, edge_index, Wl1, Wr1, b1, Wl2, Wr2, b2):
    src = edge_index[0].astype(jnp.int32)
    dst = edge_index[1].astype(jnp.int32)
    pad = E_PAD - N_EDGES
    ar = jnp.arange(pad, dtype=jnp.int32)
    src_p = jnp.concatenate([src, ar % N_NODES]).reshape(NW, CPW, CHUNK)
    dst_p = jnp.concatenate([dst, N_NODES + ar % (N_PAD - N_NODES)]
                            ).reshape(NW, CPW, CHUNK)
    agg1, cnt = _sc_agg_cnt(x, src_p, dst_p)
    c0 = cnt[0].reshape(N_PAD, 1)
    c1 = cnt[1].reshape(N_PAD, 1)
    h = _dense(True, agg1[0], agg1[1], c0, c1, x, Wl1, Wr1,
               b1.reshape(1, D))
    (agg2,) = _sc_agg(h, src_p, dst_p)
    out = _dense(False, agg2[0], agg2[1], c0, c1, h, Wl2, Wr2,
                 b2.reshape(1, D))
    return out


# speedup vs baseline: 1.1451x; 1.0104x over previous
import functools

import jax
import jax.numpy as jnp
from jax import lax
from jax.experimental import pallas as pl
from jax.experimental.pallas import tpu as pltpu
from jax.experimental.pallas import tpu_sc as plsc

N_NODES = 10000
N_EDGES = 320000
D = 128

NC = 2
NS = 16
NW = NC * NS
CHUNK = 128
CPW = 80
CPW_H = CPW // 2
EPW = CPW * CHUNK
E_PAD = NW * EPW
N_PAD = 10240
RPT = N_PAD // NS
BR = 1024
GRID = N_PAD // BR


def _make_sc_agg(compute_cnt):
    mesh = plsc.VectorSubcoreMesh(core_axis_name="c", subcore_axis_name="s")
    out_type = [jax.ShapeDtypeStruct((NC, N_PAD, D), jnp.float32)]
    if compute_cnt:
        out_type.append(jax.ShapeDtypeStruct((NC, N_PAD), jnp.float32))

    scratch = [
        pltpu.VMEM((CPW_H, CHUNK), jnp.int32),
        pltpu.VMEM((CPW_H, CHUNK), jnp.int32),
        pltpu.VMEM((2, CHUNK, D), jnp.float32),
        pltpu.VMEM((CHUNK,), jnp.float32),
        pltpu.VMEM_SHARED((N_PAD, D), jnp.float32),
        pltpu.VMEM_SHARED((N_PAD,), jnp.float32),
        pltpu.SemaphoreType.DMA,
        pltpu.SemaphoreType.DMA,
        pltpu.SemaphoreType.DMA,
    ]

    def body(x_hbm, src_hbm, dst_hbm, *rest):
        if compute_cnt:
            out_hbm, cnt_hbm = rest[0], rest[1]
            refs = rest[2:]
        else:
            out_hbm = rest[0]
            refs = rest[1:]
        src_v, dst_v, rows_v, ones_v, agg_sh, cnt_sh, sem0, sem1, csem = refs
        sems = (sem0, sem1)
        c = lax.axis_index("c")
        s = lax.axis_index("s")
        wid = c * NS + s
        rb = s * RPT
        zv = jnp.zeros((16,), jnp.float32)

        def zrow_body(i, carry):
            for k in range(D // 16):
                rows_v[0, i, pl.ds(k * 16, 16)] = zv
            return carry

        lax.fori_loop(0, CHUNK, zrow_body, 0)
        for r in range(RPT // CHUNK):
            pltpu.sync_copy(rows_v.at[0],
                            agg_sh.at[pl.ds(rb + r * CHUNK, CHUNK)])
        if compute_cnt:
            for i in range(CHUNK // 16):
                ones_v[pl.ds(i * 16, 16)] = jnp.full((16,), 1.0, jnp.float32)
            for r in range(RPT // CHUNK):
                pltpu.sync_copy(rows_v.at[0, 0],
                                cnt_sh.at[pl.ds(rb + r * CHUNK, CHUNK)])
        plsc.subcore_barrier()

        INNER = 10

        for half in range(2):
            pltpu.sync_copy(src_hbm.at[wid, pl.ds(half * CPW_H, CPW_H)],
                            src_v)
            pltpu.sync_copy(dst_hbm.at[wid, pl.ds(half * CPW_H, CPW_H)],
                            dst_v)
            for b in range(2):
                pltpu.async_copy(x_hbm.at[src_v.at[b]], rows_v.at[b],
                                 sems[b])

            def outer_body(o, carry):
                for jj in range(INNER):
                    j = o * INNER + jj
                    b = jj % 2
                    pltpu.make_async_copy(x_hbm.at[src_v.at[j]],
                                          rows_v.at[b], sems[b]).wait()
                    pltpu.sync_copy(rows_v.at[b], agg_sh.at[dst_v.at[j]],
                                    add=True)
                    if compute_cnt:
                        pltpu.async_copy(ones_v, cnt_sh.at[dst_v.at[j]],
                                         csem, add=True)
                    jn = jnp.minimum(j + 2, CPW_H - 1)
                    pltpu.async_copy(x_hbm.at[src_v.at[jn]], rows_v.at[b],
                                     sems[b])
                return carry

            lax.fori_loop(0, CPW_H // INNER, outer_body, 0)
            for b in range(2):
                pltpu.make_async_copy(x_hbm.at[src_v.at[0]], rows_v.at[b],
                                      sems[b]).wait()
            if compute_cnt:
                for _ in range(CPW_H):
                    pltpu.make_async_copy(ones_v, cnt_sh.at[dst_v.at[0]],
                                          csem).wait()
        plsc.subcore_barrier()
        pltpu.sync_copy(agg_sh.at[pl.ds(rb, RPT)], out_hbm.at[c, pl.ds(rb, RPT)])
        if compute_cnt:
            pltpu.sync_copy(cnt_sh.at[pl.ds(rb, RPT)],
                            cnt_hbm.at[c, pl.ds(rb, RPT)])

    return pl.kernel(body, out_type=out_type, mesh=mesh,
                     scratch_types=scratch)


_sc_agg_cnt = _make_sc_agg(True)
_sc_agg = _make_sc_agg(False)


_row_spec = pl.BlockSpec((BR, D), lambda i: (i, 0))
_inv_spec = pl.BlockSpec((BR,), lambda i: (i,))
_mat_spec = pl.BlockSpec((D, D), lambda i: (0, 0))
_bias_spec = pl.BlockSpec((1, D), lambda i: (0, 0))
_out_shape = jax.ShapeDtypeStruct((N_PAD, D), jnp.float32)


def _dense_body(relu, p0, p1, inv, x, wl, wr, b, o):
    mean = (p0[...] + p1[...]) * inv[...][:, None]
    h = (lax.dot_general(mean, wl[...], (((1,), (1,)), ((), ())),
                         preferred_element_type=jnp.float32)
         + lax.dot_general(x[...], wr[...], (((1,), (1,)), ((), ())),
                           preferred_element_type=jnp.float32)
         + b[...])
    if relu:
        h = jnp.maximum(h, 0.0)
    o[...] = h


def _dense(relu, p0, p1, inv, x, wl, wr, b):
    return pl.pallas_call(
        functools.partial(_dense_body, relu),
        grid=(GRID,),
        in_specs=[_row_spec, _row_spec, _inv_spec, _row_spec,
                  _mat_spec, _mat_spec, _bias_spec],
        out_specs=_row_spec, out_shape=_out_shape,
    )(p0, p1, inv, x, wl, wr, b)


def kernel(x, edge_index, Wl1, Wr1, b1, Wl2, Wr2, b2):
    src = edge_index[0].astype(jnp.int32)
    dst = edge_index[1].astype(jnp.int32)
    pad = E_PAD - N_EDGES
    ar = jnp.arange(pad, dtype=jnp.int32)
    src_p = jnp.concatenate([src, ar % N_NODES]).reshape(NW, CPW, CHUNK)
    dst_p = jnp.concatenate([dst, N_NODES + ar % (N_PAD - N_NODES)]
                            ).reshape(NW, CPW, CHUNK)
    xp = jnp.pad(x, ((0, N_PAD - N_NODES), (0, 0)))
    agg1, cnt = _sc_agg_cnt(xp, src_p, dst_p)
    inv = 1.0 / jnp.maximum(cnt[0] + cnt[1], 1.0)
    h = _dense(True, agg1[0], agg1[1], inv, xp, Wl1, Wr1,
               b1.reshape(1, D))
    (agg2,) = _sc_agg(h, src_p, dst_p)
    out = _dense(False, agg2[0], agg2[1], inv, h, Wl2, Wr2,
                 b2.reshape(1, D))
    return out[:N_NODES]
